# baseline (speedup 1.0000x reference)
import functools

import jax
import jax.numpy as jnp
from jax import lax
from jax.experimental import pallas as pl
from jax.experimental.pallas import tpu as pltpu
from jax.experimental.pallas import tpu_sc as plsc

E = 16
K = 2
D = 1024
H = 4096
RH = 1024
B_, S_ = 2, 2048
T = B_ * S_

BM_A = 256
BM_R = 1024
BM = 256
CAP = K * T + E * BM
NB = CAP // BM

NC, NS, L = 2, 16, 16
NW = NC * NS


def _router_body(x_ref, rw_ref, bias_ref, gates_ref, sel_ref):
    cdims = (((1,), (1,)), ((), ()))
    logits = lax.dot_general(x_ref[...], rw_ref[...], cdims,
                             preferred_element_type=jnp.float32)
    scores = jax.nn.sigmoid(logits)
    selsc = scores + bias_ref[0, :]
    iota = lax.broadcasted_iota(jnp.int32, selsc.shape, 1)
    m1 = jnp.max(selsc, axis=-1, keepdims=True)
    i1 = jnp.min(jnp.where(selsc == m1, iota, E), axis=-1, keepdims=True)
    sel2 = jnp.where(iota == i1, -jnp.inf, selsc)
    m2 = jnp.max(sel2, axis=-1, keepdims=True)
    i2 = jnp.min(jnp.where(sel2 == m2, iota, E), axis=-1, keepdims=True)
    selmask = (iota == i1) | (iota == i2)
    gates_ref[...] = jnp.where(selmask, scores, 0.0)
    sel_ref[...] = selmask.astype(jnp.float32)


def _router(x2d, rw, bias2):
    return pl.pallas_call(
        _router_body,
        grid=(T // BM_R,),
        in_specs=[
            pl.BlockSpec((BM_R, D), lambda i: (i, 0)),
            pl.BlockSpec((E, D), lambda i: (0, 0)),
            pl.BlockSpec((1, E), lambda i: (0, 0)),
        ],
        out_specs=[
            pl.BlockSpec((BM_R, E), lambda i: (i, 0)),
            pl.BlockSpec((BM_R, E), lambda i: (i, 0)),
        ],
        out_shape=[
            jax.ShapeDtypeStruct((T, E), jnp.float32),
            jax.ShapeDtypeStruct((T, E), jnp.float32),
        ],
    )(x2d, rw, bias2)


RW_D = T // NW
CD = 32
NCHD = RW_D // CD
NBUF = 3


@functools.partial(
    pl.kernel,
    mesh=plsc.VectorSubcoreMesh(core_axis_name="c", subcore_axis_name="s"),
    out_type=jax.ShapeDtypeStruct((CAP, D), jnp.float32),
    scratch_types=(
        [pltpu.VMEM((NCHD, CD), jnp.int32) for _ in range(2)]
        + [pltpu.VMEM((CD, D), jnp.float32) for _ in range(NBUF)]
        + [pltpu.SemaphoreType.DMA for _ in range(3 * NBUF)]
    ),
)
def _sc_dispatch(x_hbm, p0_hbm, p1_hbm, xs_hbm,
                 p0v, p1v, xb0, xb1, xb2,
                 ls0, ls1, ls2, s00, s01, s02, s10, s11, s12):
    wid = lax.axis_index("s") * NC + lax.axis_index("c")
    base = wid * RW_D
    xbuf = (xb0, xb1, xb2)
    lsem = (ls0, ls1, ls2)
    s0sem = (s00, s01, s02)
    s1sem = (s10, s11, s12)
    pltpu.sync_copy(p0_hbm.at[wid], p0v)
    pltpu.sync_copy(p1_hbm.at[wid], p1v)

    lcp = [None] * NCHD
    scp0 = [None] * NCHD
    scp1 = [None] * NCHD

    def issue_load(c):
        j = c % NBUF
        if c >= NBUF:
            scp0[c - NBUF].wait()
            scp1[c - NBUF].wait()
        lcp[c] = pltpu.async_copy(
            x_hbm.at[pl.ds(base + c * CD, CD), :], xbuf[j], lsem[j])

    issue_load(0)
    if NCHD > 1:
        issue_load(1)
    for c in range(NCHD):
        if c + 2 < NCHD:
            issue_load(c + 2)
        lcp[c].wait()
        j = c % NBUF
        scp0[c] = pltpu.async_copy(xbuf[j], xs_hbm.at[p0v.at[c]], s0sem[j])
        scp1[c] = pltpu.async_copy(xbuf[j], xs_hbm.at[p1v.at[c]], s1sem[j])
    for c in range(max(0, NCHD - NBUF), NCHD):
        scp0[c].wait()
        scp1[c].wait()


def _shared_body(x_ref, w1_ref, w2_ref, w3_ref, shared_ref):
    xb = x_ref[...]
    cdims = (((1,), (1,)), ((), ()))
    a = lax.dot_general(xb, w1_ref[...], cdims, preferred_element_type=jnp.float32)
    b = lax.dot_general(xb, w2_ref[...], cdims, preferred_element_type=jnp.float32)
    hsw = (a * jax.nn.sigmoid(a)) * b
    shared_ref[...] = lax.dot_general(hsw, w3_ref[...], cdims,
                                      preferred_element_type=jnp.float32)


def _shared(x2d, w1, w2, w3):
    nt = x2d.shape[0]
    return pl.pallas_call(
        _shared_body,
        grid=(nt // BM_A,),
        in_specs=[
            pl.BlockSpec((BM_A, D), lambda i: (i, 0)),
            pl.BlockSpec((H, D), lambda i: (0, 0)),
            pl.BlockSpec((H, D), lambda i: (0, 0)),
            pl.BlockSpec((D, H), lambda i: (0, 0)),
        ],
        out_specs=pl.BlockSpec((BM_A, D), lambda i: (i, 0)),
        out_shape=jax.ShapeDtypeStruct((nt, D), jnp.float32),
    )(x2d, w1, w2, w3)


def _grouped_body(meta_ref, xs_ref, w1_ref, w2_ref, w3_ref, buf_ref):
    i = pl.program_id(0)

    @pl.when(i < meta_ref[NB])
    def _():
        xb = xs_ref[...]
        cdims = (((1,), (1,)), ((), ()))
        a = lax.dot_general(xb, w1_ref[0], cdims, preferred_element_type=jnp.float32)
        b = lax.dot_general(xb, w2_ref[0], cdims, preferred_element_type=jnp.float32)
        hsw = (a * jax.nn.sigmoid(a)) * b
        buf_ref[...] = lax.dot_general(hsw, w3_ref[0], cdims,
                                       preferred_element_type=jnp.float32)


def _grouped(meta, xs, rw1, rw2, rw3):
    grid_spec = pltpu.PrefetchScalarGridSpec(
        num_scalar_prefetch=1,
        grid=(NB,),
        in_specs=[
            pl.BlockSpec((BM, D), lambda i, m: (i, 0)),
            pl.BlockSpec((1, RH, D), lambda i, m: (m[i], 0, 0)),
            pl.BlockSpec((1, RH, D), lambda i, m: (m[i], 0, 0)),
            pl.BlockSpec((1, D, RH), lambda i, m: (m[i], 0, 0)),
        ],
        out_specs=pl.BlockSpec((BM, D), lambda i, m: (i, 0)),
    )
    return pl.pallas_call(
        _grouped_body,
        grid_spec=grid_spec,
        out_shape=jax.ShapeDtypeStruct((CAP, D), jnp.float32),
    )(meta, xs, rw1, rw2, rw3)


TH = T // 2
RW_C = TH // NW
CC = 32


@functools.partial(
    pl.kernel,
    mesh=plsc.VectorSubcoreMesh(core_axis_name="c", subcore_axis_name="s"),
    out_type=jax.ShapeDtypeStruct((TH, D), jnp.float32),
    scratch_types=[
        pltpu.VMEM((CC,), jnp.int32),
        pltpu.VMEM((CC,), jnp.int32),
        pltpu.VMEM((CC, L), jnp.float32),
        pltpu.VMEM((CC, L), jnp.float32),
        pltpu.VMEM((CC, D), jnp.float32),
        pltpu.VMEM((CC, D), jnp.float32),
        pltpu.VMEM((CC, D), jnp.float32),
        pltpu.SemaphoreType.DMA,
        pltpu.SemaphoreType.DMA,
        pltpu.SemaphoreType.DMA,
    ],
)
def _sc_combine(sh_hbm, buf_hbm, p0_hbm, p1_hbm, g0_hbm, g1_hbm, out_hbm,
                i0, i1, g0v, g1v, r0, r1, shv, sem0, sem1, sem2):
    wid = lax.axis_index("s") * NC + lax.axis_index("c")
    base = wid * RW_C
    for c in range(RW_C // CC):
        b0 = base + c * CC
        pltpu.sync_copy(p0_hbm.at[pl.ds(b0, CC)], i0)
        pltpu.sync_copy(p1_hbm.at[pl.ds(b0, CC)], i1)
        pltpu.sync_copy(g0_hbm.at[pl.ds(b0, CC), :], g0v)
        pltpu.sync_copy(g1_hbm.at[pl.ds(b0, CC), :], g1v)
        cp0 = pltpu.async_copy(buf_hbm.at[i0], r0, sem0)
        cp1 = pltpu.async_copy(buf_hbm.at[i1], r1, sem1)
        cp2 = pltpu.async_copy(sh_hbm.at[pl.ds(b0, CC)], shv, sem2)
        cp0.wait()
        cp1.wait()
        cp2.wait()

        def row_body(r, carry):
            ga = g0v[r, :]
            gb = g1v[r, :]
            for cc in range(D // L):
                sl = pl.ds(cc * L, L)
                shv[r, sl] = shv[r, sl] + ga * r0[r, sl] + gb * r1[r, sl]
            return carry

        lax.fori_loop(0, CC, row_body, 0)
        pltpu.sync_copy(shv, out_hbm.at[pl.ds(b0, CC), :])


def kernel(x, shared_w1, shared_w2, shared_w3, routed_w1, routed_w2,
           routed_w3, router_w, expert_bias):
    x2d = x.reshape(T, D)

    gates, sel = _router(x2d, router_w, expert_bias.reshape(1, E))

    m = sel > 0.5
    mi = m.astype(jnp.int32)
    rank = jnp.cumsum(mi, axis=0) - mi
    counts = jnp.sum(mi, axis=0)
    padded = ((counts + BM - 1) // BM) * BM
    cum = jnp.cumsum(padded)
    off = cum - padded
    total = cum[-1]
    p = off[None, :] + rank

    iota = jnp.arange(E, dtype=jnp.int32)[None, :]
    e0 = jnp.min(jnp.where(m, iota, E), axis=1)
    e1 = jnp.max(jnp.where(m, iota, -1), axis=1)
    oh0 = iota == e0[:, None]
    oh1 = iota == e1[:, None]
    p0 = jnp.sum(jnp.where(oh0, p, 0), axis=1).astype(jnp.int32)
    p1 = jnp.sum(jnp.where(oh1, p, 0), axis=1).astype(jnp.int32)
    g0 = jnp.sum(jnp.where(oh0, gates, 0.0), axis=1)
    g1 = jnp.sum(jnp.where(oh1, gates, 0.0), axis=1)

    n_used = (total // BM).astype(jnp.int32)
    bidx = jnp.arange(NB, dtype=jnp.int32) * BM
    be_raw = jnp.clip(
        jnp.sum((cum[None, :] <= bidx[:, None]).astype(jnp.int32), axis=1),
        0, E - 1).astype(jnp.int32)
    last_e = jnp.take(be_raw, jnp.maximum(n_used - 1, 0))
    be = jnp.where(jnp.arange(NB) < n_used, be_raw, last_e)
    meta = jnp.concatenate([be, n_used[None]])

    xs = _sc_dispatch(x2d, p0.reshape(NW, NCHD, CD), p1.reshape(NW, NCHD, CD))
    g0b = jnp.broadcast_to(g0[:, None], (T, L))
    g1b = jnp.broadcast_to(g1[:, None], (T, L))
    shared_a = _shared(x2d[:TH], shared_w1, shared_w2, shared_w3)
    buf = _grouped(meta, xs, routed_w1, routed_w2, routed_w3)
    shared_b = _shared(x2d[TH:], shared_w1, shared_w2, shared_w3)
    out_a = _sc_combine(shared_a, buf, p0[:TH], p1[:TH], g0b[:TH], g1b[:TH])
    out_b = _sc_combine(shared_b, buf, p0[TH:], p1[TH:], g0b[TH:], g1b[TH:])
    return jnp.concatenate([out_a, out_b], axis=0).reshape(B_, S_, D)


# speedup vs baseline: 1.0730x; 1.0730x over previous
import functools

import jax
import jax.numpy as jnp
from jax import lax
from jax.experimental import pallas as pl
from jax.experimental.pallas import tpu as pltpu
from jax.experimental.pallas import tpu_sc as plsc

E = 16
K = 2
D = 1024
H = 4096
RH = 1024
B_, S_ = 2, 2048
T = B_ * S_

BM_A = 256
BM_R = 1024
BM = 256
CAP = K * T + E * BM
NB = CAP // BM

NC, NS, L = 2, 16, 16
NW = NC * NS

TH = T // 2


def _router_body(x_ref, rw_ref, bias_ref, gates_ref, sel_ref):
    cdims = (((1,), (1,)), ((), ()))
    logits = lax.dot_general(x_ref[...], rw_ref[...], cdims,
                             preferred_element_type=jnp.float32)
    scores = jax.nn.sigmoid(logits)
    selsc = scores + bias_ref[0, :]
    iota = lax.broadcasted_iota(jnp.int32, selsc.shape, 1)
    m1 = jnp.max(selsc, axis=-1, keepdims=True)
    i1 = jnp.min(jnp.where(selsc == m1, iota, E), axis=-1, keepdims=True)
    sel2 = jnp.where(iota == i1, -jnp.inf, selsc)
    m2 = jnp.max(sel2, axis=-1, keepdims=True)
    i2 = jnp.min(jnp.where(sel2 == m2, iota, E), axis=-1, keepdims=True)
    selmask = (iota == i1) | (iota == i2)
    gates_ref[...] = jnp.where(selmask, scores, 0.0)
    sel_ref[...] = selmask.astype(jnp.float32)


def _router(x2d, rw, bias2):
    return pl.pallas_call(
        _router_body,
        grid=(T // BM_R,),
        in_specs=[
            pl.BlockSpec((BM_R, D), lambda i: (i, 0)),
            pl.BlockSpec((E, D), lambda i: (0, 0)),
            pl.BlockSpec((1, E), lambda i: (0, 0)),
        ],
        out_specs=[
            pl.BlockSpec((BM_R, E), lambda i: (i, 0)),
            pl.BlockSpec((BM_R, E), lambda i: (i, 0)),
        ],
        out_shape=[
            jax.ShapeDtypeStruct((T, E), jnp.float32),
            jax.ShapeDtypeStruct((T, E), jnp.float32),
        ],
    )(x2d, rw, bias2)


RW_D = T // NW
CD = 32
NCHD = RW_D // CD
NBUF = 3


@functools.partial(
    pl.kernel,
    mesh=plsc.VectorSubcoreMesh(core_axis_name="c", subcore_axis_name="s"),
    out_type=jax.ShapeDtypeStruct((CAP, D), jnp.float32),
    scratch_types=(
        [pltpu.VMEM((NCHD, CD), jnp.int32) for _ in range(2)]
        + [pltpu.VMEM((CD, D), jnp.float32) for _ in range(NBUF)]
        + [pltpu.SemaphoreType.DMA for _ in range(3 * NBUF)]
    ),
)
def _sc_dispatch(x_hbm, p0_hbm, p1_hbm, xs_hbm,
                 p0v, p1v, xb0, xb1, xb2,
                 ls0, ls1, ls2, s00, s01, s02, s10, s11, s12):
    wid = lax.axis_index("s") * NC + lax.axis_index("c")
    base = wid * RW_D
    xbuf = (xb0, xb1, xb2)
    lsem = (ls0, ls1, ls2)
    s0sem = (s00, s01, s02)
    s1sem = (s10, s11, s12)
    pltpu.sync_copy(p0_hbm.at[wid], p0v)
    pltpu.sync_copy(p1_hbm.at[wid], p1v)

    lcp = [None] * NCHD
    scp0 = [None] * NCHD
    scp1 = [None] * NCHD

    def issue_load(c):
        j = c % NBUF
        if c >= NBUF:
            scp0[c - NBUF].wait()
            scp1[c - NBUF].wait()
        lcp[c] = pltpu.async_copy(
            x_hbm.at[pl.ds(base + c * CD, CD), :], xbuf[j], lsem[j])

    issue_load(0)
    if NCHD > 1:
        issue_load(1)
    for c in range(NCHD):
        if c + 2 < NCHD:
            issue_load(c + 2)
        lcp[c].wait()
        j = c % NBUF
        scp0[c] = pltpu.async_copy(xbuf[j], xs_hbm.at[p0v.at[c]], s0sem[j])
        scp1[c] = pltpu.async_copy(xbuf[j], xs_hbm.at[p1v.at[c]], s1sem[j])
    for c in range(max(0, NCHD - NBUF), NCHD):
        scp0[c].wait()
        scp1[c].wait()


def _shared_body(x_ref, w1_ref, w2_ref, w3_ref, shared_ref):
    xb = x_ref[...]
    cdims = (((1,), (1,)), ((), ()))
    a = lax.dot_general(xb, w1_ref[...], cdims, preferred_element_type=jnp.float32)
    b = lax.dot_general(xb, w2_ref[...], cdims, preferred_element_type=jnp.float32)
    hsw = (a * jax.nn.sigmoid(a)) * b
    shared_ref[...] = lax.dot_general(hsw, w3_ref[...], cdims,
                                      preferred_element_type=jnp.float32)


def _shared_half(off_blocks):
    def call(x2d, w1, w2, w3):
        return pl.pallas_call(
            _shared_body,
            grid=(TH // BM_A,),
            in_specs=[
                pl.BlockSpec((BM_A, D), lambda i: (i + off_blocks, 0)),
                pl.BlockSpec((H, D), lambda i: (0, 0)),
                pl.BlockSpec((H, D), lambda i: (0, 0)),
                pl.BlockSpec((D, H), lambda i: (0, 0)),
            ],
            out_specs=pl.BlockSpec((BM_A, D), lambda i: (i, 0)),
            out_shape=jax.ShapeDtypeStruct((TH, D), jnp.float32),
        )(x2d, w1, w2, w3)
    return call


def _grouped_body(meta_ref, xs_ref, w1_ref, w2_ref, w3_ref, buf_ref):
    i = pl.program_id(0)

    @pl.when(i < meta_ref[NB])
    def _():
        xb = xs_ref[...]
        cdims = (((1,), (1,)), ((), ()))
        a = lax.dot_general(xb, w1_ref[0], cdims, preferred_element_type=jnp.float32)
        b = lax.dot_general(xb, w2_ref[0], cdims, preferred_element_type=jnp.float32)
        hsw = (a * jax.nn.sigmoid(a)) * b
        buf_ref[...] = lax.dot_general(hsw, w3_ref[0], cdims,
                                       preferred_element_type=jnp.float32)


def _grouped(meta, xs, rw1, rw2, rw3):
    grid_spec = pltpu.PrefetchScalarGridSpec(
        num_scalar_prefetch=1,
        grid=(NB,),
        in_specs=[
            pl.BlockSpec((BM, D), lambda i, m: (jnp.minimum(i, m[NB] - 1), 0)),
            pl.BlockSpec((1, RH, D), lambda i, m: (m[i], 0, 0)),
            pl.BlockSpec((1, RH, D), lambda i, m: (m[i], 0, 0)),
            pl.BlockSpec((1, D, RH), lambda i, m: (m[i], 0, 0)),
        ],
        out_specs=pl.BlockSpec((BM, D), lambda i, m: (jnp.minimum(i, m[NB] - 1), 0)),
    )
    return pl.pallas_call(
        _grouped_body,
        grid_spec=grid_spec,
        out_shape=jax.ShapeDtypeStruct((CAP, D), jnp.float32),
    )(meta, xs, rw1, rw2, rw3)


RW_C = TH // NW
CC = 32


def _make_combine(off):
    @functools.partial(
        pl.kernel,
        mesh=plsc.VectorSubcoreMesh(core_axis_name="c", subcore_axis_name="s"),
        out_type=jax.ShapeDtypeStruct((TH, D), jnp.float32),
        scratch_types=[
            pltpu.VMEM((CC,), jnp.int32),
            pltpu.VMEM((CC,), jnp.int32),
            pltpu.VMEM((CC, L), jnp.float32),
            pltpu.VMEM((CC, L), jnp.float32),
            pltpu.VMEM((CC, D), jnp.float32),
            pltpu.VMEM((CC, D), jnp.float32),
            pltpu.VMEM((CC, D), jnp.float32),
            pltpu.SemaphoreType.DMA,
            pltpu.SemaphoreType.DMA,
            pltpu.SemaphoreType.DMA,
        ],
    )
    def _sc_combine(sh_hbm, buf_hbm, p0_hbm, p1_hbm, g0_hbm, g1_hbm, out_hbm,
                    i0, i1, g0v, g1v, r0, r1, shv, sem0, sem1, sem2):
        wid = lax.axis_index("s") * NC + lax.axis_index("c")
        base = wid * RW_C
        for c in range(RW_C // CC):
            b0 = base + c * CC
            gb0 = off + b0
            pltpu.sync_copy(p0_hbm.at[pl.ds(gb0, CC)], i0)
            pltpu.sync_copy(p1_hbm.at[pl.ds(gb0, CC)], i1)
            pltpu.sync_copy(g0_hbm.at[pl.ds(gb0, CC), :], g0v)
            pltpu.sync_copy(g1_hbm.at[pl.ds(gb0, CC), :], g1v)
            cp0 = pltpu.async_copy(buf_hbm.at[i0], r0, sem0)
            cp1 = pltpu.async_copy(buf_hbm.at[i1], r1, sem1)
            cp2 = pltpu.async_copy(sh_hbm.at[pl.ds(b0, CC)], shv, sem2)
            cp0.wait()
            cp1.wait()
            cp2.wait()

            def row_body(r, carry):
                ga = g0v[r, :]
                gb = g1v[r, :]
                for cc in range(D // L):
                    sl = pl.ds(cc * L, L)
                    shv[r, sl] = shv[r, sl] + ga * r0[r, sl] + gb * r1[r, sl]
                return carry

            lax.fori_loop(0, CC, row_body, 0)
            pltpu.sync_copy(shv, out_hbm.at[pl.ds(b0, CC), :])
    return _sc_combine


_combine_lo = _make_combine(0)
_combine_hi = _make_combine(TH)


def kernel(x, shared_w1, shared_w2, shared_w3, routed_w1, routed_w2,
           routed_w3, router_w, expert_bias):
    x2d = x.reshape(T, D)

    gates, sel = _router(x2d, router_w, expert_bias.reshape(1, E))

    m = sel > 0.5
    mi = m.astype(jnp.int32)
    rank = jnp.cumsum(mi, axis=0) - mi
    counts = jnp.sum(mi, axis=0)
    padded = ((counts + BM - 1) // BM) * BM
    cum = jnp.cumsum(padded)
    off = cum - padded
    total = cum[-1]
    p = off[None, :] + rank

    iota = jnp.arange(E, dtype=jnp.int32)[None, :]
    e0 = jnp.min(jnp.where(m, iota, E), axis=1)
    e1 = jnp.max(jnp.where(m, iota, -1), axis=1)
    oh0 = iota == e0[:, None]
    oh1 = iota == e1[:, None]
    p0 = jnp.sum(jnp.where(oh0, p, 0), axis=1).astype(jnp.int32)
    p1 = jnp.sum(jnp.where(oh1, p, 0), axis=1).astype(jnp.int32)
    g0 = jnp.sum(jnp.where(oh0, gates, 0.0), axis=1)
    g1 = jnp.sum(jnp.where(oh1, gates, 0.0), axis=1)

    n_used = (total // BM).astype(jnp.int32)
    bidx = jnp.arange(NB, dtype=jnp.int32) * BM
    be_raw = jnp.clip(
        jnp.sum((cum[None, :] <= bidx[:, None]).astype(jnp.int32), axis=1),
        0, E - 1).astype(jnp.int32)
    last_e = jnp.take(be_raw, jnp.maximum(n_used - 1, 0))
    be = jnp.where(jnp.arange(NB) < n_used, be_raw, last_e)
    meta = jnp.concatenate([be, n_used[None]])

    xs = _sc_dispatch(x2d, p0.reshape(NW, NCHD, CD), p1.reshape(NW, NCHD, CD))
    g0b = jnp.broadcast_to(g0[:, None], (T, L))
    g1b = jnp.broadcast_to(g1[:, None], (T, L))
    shared_a = _shared_half(0)(x2d, shared_w1, shared_w2, shared_w3)
    buf = _grouped(meta, xs, routed_w1, routed_w2, routed_w3)
    shared_b = _shared_half(TH // BM_A)(x2d, shared_w1, shared_w2, shared_w3)
    out_a = _combine_lo(shared_a, buf, p0, p1, g0b, g1b)
    out_b = _combine_hi(shared_b, buf, p0, p1, g0b, g1b)
    return jnp.concatenate([out_a, out_b], axis=0).reshape(B_, S_, D)
